# Initial kernel scaffold; baseline (speedup 1.0000x reference)
import jax
import jax.numpy as jnp
from jax.experimental import pallas as pl


def kernel(x, edge_index_in, edge_weight_in, edge_index_out, edge_weight_out):
    raise NotImplementedError("write your pallas kernel here")


# speedup vs baseline: 1.5159x; 1.5159x over previous
import jax
import jax.numpy as jnp
from jax import lax
from jax.experimental import pallas as pl
from jax.experimental.pallas import tpu as pltpu
from jax.experimental.pallas import tpu_sc as plsc

N = 10000
D = 128
LAYERS = 2
NC = 2
NS = 16
NW = NC * NS
L = 16
CP = D // NW
SLICE = N * CP
EB = 6400


def _sc_body(xr, rows, cols, w, out, hs, acc, rb, cb, wb):
    t = lax.axis_index("c") * NS + lax.axis_index("s")
    e2 = rows.shape[0]
    nblk = e2 // EB

    pltpu.sync_copy(xr.at[t], hs)

    @pl.loop(0, SLICE // L)
    def _zero(i):
        acc[pl.ds(i * L, L)] = jnp.zeros((L,), jnp.float32)

    for layer in range(LAYERS):
        @pl.loop(0, nblk)
        def _blk(b):
            pltpu.sync_copy(rows.at[pl.ds(b * EB, EB)], rb)
            pltpu.sync_copy(cols.at[pl.ds(b * EB, EB)], cb)
            pltpu.sync_copy(w.at[pl.ds(b * EB, EB)], wb)

            @pl.loop(0, EB // L)
            def _grp(g):
                rv = rb[pl.ds(g * L, L)] * CP
                cv = cb[pl.ds(g * L, L)] * CP
                wv = wb[pl.ds(g * L, L)]
                for c in range(CP):
                    gathered = plsc.load_gather(hs, [cv + c])
                    plsc.addupdate_scatter(acc, [rv + c], gathered * wv)

        @pl.loop(0, SLICE // L)
        def _upd(i):
            sl = pl.ds(i * L, L)
            hs[sl] = (hs[sl] + acc[sl]) * 0.5
            acc[sl] = jnp.zeros((L,), jnp.float32)

    pltpu.sync_copy(hs, out.at[t])


def kernel(x, edge_index_in, edge_weight_in, edge_index_out, edge_weight_out):
    rows = jnp.concatenate([edge_index_in[0], edge_index_out[0]])
    cols = jnp.concatenate([edge_index_in[1], edge_index_out[1]])
    w = jnp.concatenate([edge_weight_in, edge_weight_out])

    e2 = rows.shape[0]
    pad = (-e2) % EB
    if pad:
        rows = jnp.concatenate([rows, jnp.zeros((pad,), rows.dtype)])
        cols = jnp.concatenate([cols, jnp.zeros((pad,), cols.dtype)])
        w = jnp.concatenate([w, jnp.zeros((pad,), w.dtype)])

    xr = x.reshape(N, NW, CP).transpose(1, 0, 2).reshape(NW, SLICE)

    mesh = plsc.VectorSubcoreMesh(
        core_axis_name="c", subcore_axis_name="s",
        num_cores=NC, num_subcores=NS,
    )
    out_r = pl.kernel(
        _sc_body,
        out_type=jax.ShapeDtypeStruct((NW, SLICE), jnp.float32),
        mesh=mesh,
        compiler_params=pltpu.CompilerParams(needs_layout_passes=False),
        scratch_types=[
            pltpu.VMEM((SLICE,), jnp.float32),
            pltpu.VMEM((SLICE,), jnp.float32),
            pltpu.VMEM((EB,), jnp.int32),
            pltpu.VMEM((EB,), jnp.int32),
            pltpu.VMEM((EB,), jnp.float32),
        ],
    )(xr, rows, cols, w)

    return out_r.reshape(NW, N, CP).transpose(1, 0, 2).reshape(N, D)


# speedup vs baseline: 1.7835x; 1.1766x over previous
import jax
import jax.numpy as jnp
from jax import lax
from jax.experimental import pallas as pl
from jax.experimental.pallas import tpu as pltpu
from jax.experimental.pallas import tpu_sc as plsc

N = 10000
D = 128
LAYERS = 2
NC = 2
NS = 16
NW = NC * NS
L = 16
CP = D // NW
SLICE = N * CP
EB = 6400
UNROLL = 8


def _sc_body(xr, rc, w, out, hs, acc, rc0, rc1, w0, w1, sem0, sem1):
    t = lax.axis_index("c") * NS + lax.axis_index("s")
    e2 = rc.shape[0]
    nblk = e2 // EB
    rcb = (rc0, rc1)
    wb = (w0, w1)
    sems = (sem0, sem1)

    def start(bi, ph):
        sl = pl.ds(bi * EB, EB)
        pltpu.async_copy(rc.at[sl], rcb[ph], sems[ph])
        pltpu.async_copy(w.at[sl], wb[ph], sems[ph])

    def drain(ph):
        sl = pl.ds(0, EB)
        pltpu.make_async_copy(rc.at[sl], rcb[ph], sems[ph]).wait()
        pltpu.make_async_copy(w.at[sl], wb[ph], sems[ph]).wait()

    start(0, 0)
    start(1, 1)
    pltpu.sync_copy(xr.at[t], hs)

    @pl.loop(0, SLICE // L, unroll=UNROLL)
    def _zero(i):
        acc[pl.ds(i * L, L)] = jnp.zeros((L,), jnp.float32)

    for layer in range(LAYERS):
        last = layer == LAYERS - 1

        @pl.loop(0, nblk, step=2)
        def _blk(b):
            for ph in range(2):
                drain(ph)

                @pl.loop(0, EB // L, unroll=UNROLL)
                def _grp(g):
                    v = rcb[ph][pl.ds(g * L, L)]
                    wv = wb[ph][pl.ds(g * L, L)]
                    rv4 = v & jnp.int32(0xFFFF)
                    cv4 = lax.shift_right_logical(v, 16)
                    for c in range(CP):
                        g16 = plsc.load_gather(hs, [cv4 + c])
                        plsc.addupdate_scatter(acc, [rv4 + c], g16 * wv)

                nxt = b + ph + 2
                if last:
                    @pl.when(nxt < nblk)
                    def _():
                        start(nxt, ph)
                else:
                    start(lax.rem(nxt, nblk), ph)

        @pl.loop(0, SLICE // L, unroll=UNROLL)
        def _upd(i):
            sl = pl.ds(i * L, L)
            hs[sl] = (hs[sl] + acc[sl]) * 0.5
            if not last:
                acc[sl] = jnp.zeros((L,), jnp.float32)

    pltpu.sync_copy(hs, out.at[t])


def kernel(x, edge_index_in, edge_weight_in, edge_index_out, edge_weight_out):
    rows = jnp.concatenate([edge_index_in[0], edge_index_out[0]])
    cols = jnp.concatenate([edge_index_in[1], edge_index_out[1]])
    w = jnp.concatenate([edge_weight_in, edge_weight_out])

    e2 = rows.shape[0]
    pad = (-e2) % (2 * EB)
    if pad:
        rows = jnp.concatenate([rows, jnp.zeros((pad,), rows.dtype)])
        cols = jnp.concatenate([cols, jnp.zeros((pad,), cols.dtype)])
        w = jnp.concatenate([w, jnp.zeros((pad,), w.dtype)])

    rc = (rows * CP) | lax.shift_left(cols * CP, 16)

    xr = x.reshape(N, NW, CP).transpose(1, 0, 2).reshape(NW, SLICE)

    mesh = plsc.VectorSubcoreMesh(
        core_axis_name="c", subcore_axis_name="s",
        num_cores=NC, num_subcores=NS,
    )
    out_r = pl.kernel(
        _sc_body,
        out_type=jax.ShapeDtypeStruct((NW, SLICE), jnp.float32),
        mesh=mesh,
        compiler_params=pltpu.CompilerParams(needs_layout_passes=False),
        scratch_types=[
            pltpu.VMEM((SLICE,), jnp.float32),
            pltpu.VMEM((SLICE,), jnp.float32),
            pltpu.VMEM((EB,), jnp.int32),
            pltpu.VMEM((EB,), jnp.int32),
            pltpu.VMEM((EB,), jnp.float32),
            pltpu.VMEM((EB,), jnp.float32),
            pltpu.SemaphoreType.DMA,
            pltpu.SemaphoreType.DMA,
        ],
    )(xr, rc, w)

    return out_r.reshape(NW, N, CP).transpose(1, 0, 2).reshape(N, D)
